# baseline (speedup 1.0000x reference)
import jax
import jax.numpy as jnp
from jax.experimental import pallas as pl

_THR = 0.005
_EPS = 1e-8
_NEG = -1e30


def _fused_body(x_ref, keys_ref, values_ref, init_ref, o_ref):
    x = x_ref[...]
    m = jnp.max(x, axis=-1, keepdims=True)
    e = jnp.exp(x - m)
    probs = e / jnp.sum(e, axis=-1, keepdims=True)
    qn = probs / jnp.maximum(
        jnp.sqrt(jnp.sum(probs * probs, axis=-1, keepdims=True)), _EPS)

    k = keys_ref[...]
    kn = k / jnp.maximum(
        jnp.sqrt(jnp.sum(k * k, axis=-1, keepdims=True)), _EPS)

    sim = jax.lax.dot_general(
        qn, kn, (((1,), (1,)), ((), ())),
        preferred_element_type=jnp.float32)

    masked = jnp.where(sim > _THR, sim, _NEG)
    maxv = jnp.max(masked, axis=-1, keepdims=True)
    has = maxv >= _THR
    safe = jnp.where(has, masked, 0.0)
    sm = jnp.max(safe, axis=-1, keepdims=True)
    se = jnp.exp(safe - sm)
    w = se / jnp.sum(se, axis=-1, keepdims=True)

    retrieved = jnp.dot(w, values_ref[...],
                        preferred_element_type=jnp.float32)
    o_ref[...] = jnp.where(has, retrieved, init_ref[...])


def kernel(output, keys, values, init_prompt):
    B, C = output.shape
    K, D = values.shape
    Cp = (C + 127) // 128 * 128
    Kp = (K + 127) // 128 * 128
    Bb = 256

    outp = jnp.pad(output, ((0, 0), (0, Cp - C)), constant_values=_NEG)
    keysp = jnp.pad(keys, ((0, Kp - K), (0, Cp - C)))
    valsp = jnp.pad(values, ((0, Kp - K), (0, 0)))
    initp = init_prompt.reshape(1, D)

    return pl.pallas_call(
        _fused_body,
        grid=(B // Bb,),
        in_specs=[
            pl.BlockSpec((Bb, Cp), lambda i: (i, 0)),
            pl.BlockSpec((Kp, Cp), lambda i: (0, 0)),
            pl.BlockSpec((Kp, D), lambda i: (0, 0)),
            pl.BlockSpec((1, D), lambda i: (0, 0)),
        ],
        out_specs=pl.BlockSpec((Bb, D), lambda i: (i, 0)),
        out_shape=jax.ShapeDtypeStruct((B, D), jnp.float32),
    )(outp, keysp, valsp, initp)


# speedup vs baseline: 1.4792x; 1.4792x over previous
import jax
import jax.numpy as jnp
from jax.experimental import pallas as pl

_THR = 0.005
_EPS = 1e-8
_NEG = -1e30


def _fused_body(x_ref, keys_ref, values_ref, init_ref, o_ref):
    x = x_ref[...]
    m = jnp.max(x, axis=-1, keepdims=True)
    e = jnp.exp(x - m)
    probs = e / jnp.sum(e, axis=-1, keepdims=True)
    qn = probs / jnp.maximum(
        jnp.sqrt(jnp.sum(probs * probs, axis=-1, keepdims=True)), _EPS)

    k = keys_ref[...]
    kn = k / jnp.maximum(
        jnp.sqrt(jnp.sum(k * k, axis=-1, keepdims=True)), _EPS)

    sim = jax.lax.dot_general(
        qn, kn, (((1,), (1,)), ((), ())),
        preferred_element_type=jnp.float32)

    masked = jnp.where(sim > _THR, sim, _NEG)
    maxv = jnp.max(masked, axis=-1, keepdims=True)
    has = maxv >= _THR
    safe = jnp.where(has, masked, 0.0)
    sm = jnp.max(safe, axis=-1, keepdims=True)
    se = jnp.exp(safe - sm)
    w = se / jnp.sum(se, axis=-1, keepdims=True)

    retrieved = jnp.dot(w, values_ref[...],
                        preferred_element_type=jnp.float32)
    o_ref[...] = jnp.where(has, retrieved, init_ref[...])


def kernel(output, keys, values, init_prompt):
    B, C = output.shape
    K, D = values.shape
    Bb = 256

    initp = init_prompt.reshape(1, D)

    return pl.pallas_call(
        _fused_body,
        grid=(B // Bb,),
        in_specs=[
            pl.BlockSpec((Bb, C), lambda i: (i, 0)),
            pl.BlockSpec((K, C), lambda i: (0, 0)),
            pl.BlockSpec((K, D), lambda i: (0, 0)),
            pl.BlockSpec((1, D), lambda i: (0, 0)),
        ],
        out_specs=pl.BlockSpec((Bb, D), lambda i: (i, 0)),
        out_shape=jax.ShapeDtypeStruct((B, D), jnp.float32),
    )(output, keys, values, initp)


# speedup vs baseline: 1.5661x; 1.0588x over previous
import jax
import jax.numpy as jnp
from jax.experimental import pallas as pl
from jax.experimental.pallas import tpu as pltpu

_THR = 0.005
_EPS = 1e-8
_NEG = -1e30


def _fused_body(x_ref, keys_ref, values_ref, init_ref, o_ref, kn_ref):
    @pl.when(pl.program_id(0) == 0)
    def _():
        k = keys_ref[...]
        kn_ref[...] = k / jnp.maximum(
            jnp.sqrt(jnp.sum(k * k, axis=-1, keepdims=True)), _EPS)

    x = x_ref[...]
    m = jnp.max(x, axis=-1, keepdims=True)
    e = jnp.exp(x - m)
    probs = e / jnp.sum(e, axis=-1, keepdims=True)
    qn = probs / jnp.maximum(
        jnp.sqrt(jnp.sum(probs * probs, axis=-1, keepdims=True)), _EPS)

    sim = jax.lax.dot_general(
        qn, kn_ref[...], (((1,), (1,)), ((), ())),
        preferred_element_type=jnp.float32)

    masked = jnp.where(sim > _THR, sim, _NEG)
    maxv = jnp.max(masked, axis=-1, keepdims=True)
    has = maxv >= _THR
    safe = jnp.where(has, masked, 0.0)
    sm = jnp.max(safe, axis=-1, keepdims=True)
    se = jnp.exp(safe - sm)
    w = se / jnp.sum(se, axis=-1, keepdims=True)

    retrieved = jnp.dot(w, values_ref[...],
                        preferred_element_type=jnp.float32)
    o_ref[...] = jnp.where(has, retrieved, init_ref[...])


def kernel(output, keys, values, init_prompt):
    B, C = output.shape
    K, D = values.shape
    Bb = 256

    initp = init_prompt.reshape(1, D)

    return pl.pallas_call(
        _fused_body,
        grid=(B // Bb,),
        in_specs=[
            pl.BlockSpec((Bb, C), lambda i: (i, 0)),
            pl.BlockSpec((K, C), lambda i: (0, 0)),
            pl.BlockSpec((K, D), lambda i: (0, 0)),
            pl.BlockSpec((1, D), lambda i: (0, 0)),
        ],
        out_specs=pl.BlockSpec((Bb, D), lambda i: (i, 0)),
        out_shape=jax.ShapeDtypeStruct((B, D), jnp.float32),
        scratch_shapes=[pltpu.VMEM((K, C), jnp.float32)],
    )(output, keys, values, initp)


# speedup vs baseline: 1.5691x; 1.0019x over previous
import jax
import jax.numpy as jnp
from jax.experimental import pallas as pl
from jax.experimental.pallas import tpu as pltpu

_THR = 0.005
_EPS = 1e-8
_NEG = -1e30


def _fused_body(x_ref, keys_ref, values_ref, init_ref, o_ref, kn_ref, vb_ref):
    @pl.when(pl.program_id(0) == 0)
    def _():
        k = keys_ref[...]
        kn = k / jnp.maximum(
            jnp.sqrt(jnp.sum(k * k, axis=-1, keepdims=True)), _EPS)
        kn_ref[...] = kn.astype(jnp.bfloat16)
        vb_ref[...] = values_ref[...].astype(jnp.bfloat16)

    x = x_ref[...]
    m = jnp.max(x, axis=-1, keepdims=True)
    e = jnp.exp(x - m)
    probs = e / jnp.sum(e, axis=-1, keepdims=True)
    qn = probs / jnp.maximum(
        jnp.sqrt(jnp.sum(probs * probs, axis=-1, keepdims=True)), _EPS)

    sim = jax.lax.dot_general(
        qn.astype(jnp.bfloat16), kn_ref[...], (((1,), (1,)), ((), ())),
        preferred_element_type=jnp.float32)

    masked = jnp.where(sim > _THR, sim, _NEG)
    maxv = jnp.max(masked, axis=-1, keepdims=True)
    has = maxv >= _THR
    safe = jnp.where(has, masked, 0.0)
    sm = jnp.max(safe, axis=-1, keepdims=True)
    se = jnp.exp(safe - sm)
    ssum = jnp.sum(se, axis=-1, keepdims=True)

    retrieved = jnp.dot(se.astype(jnp.bfloat16), vb_ref[...],
                        preferred_element_type=jnp.float32) / ssum
    o_ref[...] = jnp.where(has, retrieved, init_ref[...])


def kernel(output, keys, values, init_prompt):
    B, C = output.shape
    K, D = values.shape
    Bb = 256

    initp = init_prompt.reshape(1, D)

    return pl.pallas_call(
        _fused_body,
        grid=(B // Bb,),
        in_specs=[
            pl.BlockSpec((Bb, C), lambda i: (i, 0)),
            pl.BlockSpec((K, C), lambda i: (0, 0)),
            pl.BlockSpec((K, D), lambda i: (0, 0)),
            pl.BlockSpec((1, D), lambda i: (0, 0)),
        ],
        out_specs=pl.BlockSpec((Bb, D), lambda i: (i, 0)),
        out_shape=jax.ShapeDtypeStruct((B, D), jnp.float32),
        scratch_shapes=[pltpu.VMEM((K, C), jnp.bfloat16),
                        pltpu.VMEM((K, D), jnp.bfloat16)],
    )(output, keys, values, initp)


# speedup vs baseline: 1.8561x; 1.1829x over previous
import jax
import jax.numpy as jnp
from jax.experimental import pallas as pl
from jax.experimental.pallas import tpu as pltpu

_THR = 0.005
_EPS = 1e-8


def _fused_body(x_ref, keys_ref, values_ref, init_ref, o_ref, kn_ref, vb_ref):
    @pl.when(pl.program_id(0) == 0)
    def _():
        k = keys_ref[...]
        kn = k / jnp.maximum(
            jnp.sqrt(jnp.sum(k * k, axis=-1, keepdims=True)), _EPS)
        kn_ref[...] = kn.astype(jnp.bfloat16)
        vb_ref[...] = values_ref[...].astype(jnp.bfloat16)

    x = x_ref[...]
    m = jnp.max(x, axis=-1, keepdims=True)
    e = jnp.exp(x - m)
    rn = jax.lax.rsqrt(jnp.sum(e * e, axis=-1, keepdims=True))

    u = jax.lax.dot_general(
        e.astype(jnp.bfloat16), kn_ref[...], (((1,), (1,)), ((), ())),
        preferred_element_type=jnp.float32)
    sim = u * rn

    se = jnp.where(sim > _THR, jnp.exp(sim), 0.0)
    ssum = jnp.sum(se, axis=-1, keepdims=True)

    retrieved = jnp.dot(se.astype(jnp.bfloat16), vb_ref[...],
                        preferred_element_type=jnp.float32) / ssum
    o_ref[...] = jnp.where(ssum > 0.0, retrieved, init_ref[...])


def kernel(output, keys, values, init_prompt):
    B, C = output.shape
    K, D = values.shape
    Bb = 256

    initp = init_prompt.reshape(1, D)

    return pl.pallas_call(
        _fused_body,
        grid=(B // Bb,),
        in_specs=[
            pl.BlockSpec((Bb, C), lambda i: (i, 0)),
            pl.BlockSpec((K, C), lambda i: (0, 0)),
            pl.BlockSpec((K, D), lambda i: (0, 0)),
            pl.BlockSpec((1, D), lambda i: (0, 0)),
        ],
        out_specs=pl.BlockSpec((Bb, D), lambda i: (i, 0)),
        out_shape=jax.ShapeDtypeStruct((B, D), jnp.float32),
        scratch_shapes=[pltpu.VMEM((K, C), jnp.bfloat16),
                        pltpu.VMEM((K, D), jnp.bfloat16)],
    )(output, keys, values, initp)


# speedup vs baseline: 2.0124x; 1.0842x over previous
import jax
import jax.numpy as jnp
from jax.experimental import pallas as pl
from jax.experimental.pallas import tpu as pltpu

_THR = 0.005
_EPS = 1e-8


def _fused_body(x_ref, keys_ref, values_ref, init_ref, o_ref, kn_ref, vb_ref):
    @pl.when(pl.program_id(0) == 0)
    def _():
        k = keys_ref[...]
        kn = k / jnp.maximum(
            jnp.sqrt(jnp.sum(k * k, axis=-1, keepdims=True)), _EPS)
        kn_ref[...] = kn.astype(jnp.bfloat16)
        vb_ref[...] = values_ref[...].astype(jnp.bfloat16)

    x = x_ref[...]
    m = jnp.max(x, axis=-1, keepdims=True)
    e = jnp.exp(x - m)
    rn = jax.lax.rsqrt(jnp.sum(e * e, axis=-1, keepdims=True))

    u = jax.lax.dot_general(
        e.astype(jnp.bfloat16), kn_ref[...], (((1,), (1,)), ((), ())),
        preferred_element_type=jnp.float32)
    sim = u * rn

    se = jnp.where(sim > _THR, jnp.exp(sim), 0.0)
    ssum = jnp.sum(se, axis=-1, keepdims=True)

    retrieved = jnp.dot(se.astype(jnp.bfloat16), vb_ref[...],
                        preferred_element_type=jnp.float32) / ssum
    o_ref[...] = jnp.where(ssum > 0.0, retrieved, init_ref[...])


def kernel(output, keys, values, init_prompt):
    B, C = output.shape
    K, D = values.shape
    Bb = 512

    initp = init_prompt.reshape(1, D)

    return pl.pallas_call(
        _fused_body,
        grid=(B // Bb,),
        in_specs=[
            pl.BlockSpec((Bb, C), lambda i: (i, 0)),
            pl.BlockSpec((K, C), lambda i: (0, 0)),
            pl.BlockSpec((K, D), lambda i: (0, 0)),
            pl.BlockSpec((1, D), lambda i: (0, 0)),
        ],
        out_specs=pl.BlockSpec((Bb, D), lambda i: (i, 0)),
        out_shape=jax.ShapeDtypeStruct((B, D), jnp.float32),
        scratch_shapes=[pltpu.VMEM((K, C), jnp.bfloat16),
                        pltpu.VMEM((K, D), jnp.bfloat16)],
    )(output, keys, values, initp)


# speedup vs baseline: 2.0275x; 1.0075x over previous
import jax
import jax.numpy as jnp
from jax.experimental import pallas as pl
from jax.experimental.pallas import tpu as pltpu

_THR = 0.005
_EPS = 1e-8


def _fused_body(x_ref, keys_ref, values_ref, init_ref, o_ref, kn_ref, vb_ref):
    @pl.when(pl.program_id(0) == 0)
    def _():
        k = keys_ref[...]
        kn = k / jnp.maximum(
            jnp.sqrt(jnp.sum(k * k, axis=-1, keepdims=True)), _EPS)
        kn_ref[...] = kn.astype(jnp.bfloat16)
        vb_ref[...] = values_ref[...].astype(jnp.bfloat16)

    x = x_ref[...]
    m = jnp.max(x, axis=-1, keepdims=True)
    e = jnp.exp(x - m)
    rn = jax.lax.rsqrt(jnp.sum(e * e, axis=-1, keepdims=True))

    u = jax.lax.dot_general(
        e.astype(jnp.bfloat16), kn_ref[...], (((1,), (1,)), ((), ())),
        preferred_element_type=jnp.float32)
    sim = u * rn

    se = jnp.where(sim > _THR, jnp.exp(sim), 0.0)
    ssum = jnp.sum(se, axis=-1, keepdims=True)

    retrieved = jnp.dot(se.astype(jnp.bfloat16), vb_ref[...],
                        preferred_element_type=jnp.float32) / ssum
    o_ref[...] = jnp.where(ssum > 0.0, retrieved, init_ref[...])


def kernel(output, keys, values, init_prompt):
    B, C = output.shape
    K, D = values.shape
    Bb = 1024

    initp = init_prompt.reshape(1, D)

    return pl.pallas_call(
        _fused_body,
        grid=(B // Bb,),
        in_specs=[
            pl.BlockSpec((Bb, C), lambda i: (i, 0)),
            pl.BlockSpec((K, C), lambda i: (0, 0)),
            pl.BlockSpec((K, D), lambda i: (0, 0)),
            pl.BlockSpec((1, D), lambda i: (0, 0)),
        ],
        out_specs=pl.BlockSpec((Bb, D), lambda i: (i, 0)),
        out_shape=jax.ShapeDtypeStruct((B, D), jnp.float32),
        scratch_shapes=[pltpu.VMEM((K, C), jnp.bfloat16),
                        pltpu.VMEM((K, D), jnp.bfloat16)],
    )(output, keys, values, initp)
